# baseline (speedup 1.0000x reference)
import functools

import jax
import jax.numpy as jnp
from jax import lax
from jax.experimental import pallas as pl
from jax.experimental.pallas import tpu as pltpu
from jax.experimental.pallas import tpu_sc as plsc

BATCH = 16384
N_FIELDS = 26
EMB = 64
D2 = N_FIELDS * EMB
G = 4
GD = G * EMB
NC, NS = 2, 16
NW = NC * NS
CHUNK = 104
NBUF = 4
B_G_TOT = BATCH * N_FIELDS // G

SC_BATCH = 8192
B_G = SC_BATCH * N_FIELDS // G
BPW = B_G // NW
NCHUNK = BPW // CHUNK
TC_BB = 512
TC_OFF = SC_BATCH // TC_BB

_mesh = plsc.VectorSubcoreMesh(core_axis_name="c", subcore_axis_name="s")


@functools.partial(
    pl.kernel,
    mesh=_mesh,
    out_type=jax.ShapeDtypeStruct((B_G_TOT, GD), jnp.float32),
    scratch_types=(
        [pltpu.VMEM((BPW,), jnp.int32)]
        + [pltpu.VMEM((CHUNK, GD), jnp.float32) for _ in range(NBUF)]
        + [pltpu.SemaphoreType.DMA for _ in range(2 * NBUF)]
    ),
)
def _sc_embed(table_hbm, idx_hbm, out_hbm, idx_v, *bufs):
    rows = bufs[:NBUF]
    sg = bufs[NBUF:2 * NBUF]
    sw = bufs[2 * NBUF:]
    wid = lax.axis_index("s") * NC + lax.axis_index("c")
    base0 = wid * BPW

    pltpu.sync_copy(idx_hbm.at[pl.ds(base0, BPW)], idx_v)

    def start_gather(k):
        b = k % NBUF
        return pltpu.async_copy(
            table_hbm.at[idx_v.at[pl.ds(k * CHUNK, CHUNK)]],
            rows[b], sg[b])

    def start_write(k):
        b = k % NBUF
        return pltpu.async_copy(
            rows[b], out_hbm.at[pl.ds(base0 + k * CHUNK, CHUNK)],
            sw[b])

    g = {k: start_gather(k) for k in range(NBUF - 1)}
    w = {}
    for k in range(NCHUNK):
        if k + NBUF - 1 < NCHUNK:
            if k >= 1:
                w[k - 1].wait()
            g[k + NBUF - 1] = start_gather(k + NBUF - 1)
        g[k].wait()
        w[k] = start_write(k)
    for k in range(max(0, NCHUNK - NBUF), NCHUNK):
        w[k].wait()


def _tc_body(ind_ref, e_ref, difft_ref, w0t_ref, buf_ref, out_ref):
    del buf_ref
    indf = ind_ref[...].astype(jnp.float32)
    rep = jnp.dot(indf, e_ref[...], preferred_element_type=jnp.float32)
    out_ref[...] = rep * difft_ref[...] + w0t_ref[...]


def kernel(ind, ind_emb_weight):
    w = ind_emb_weight
    ind32 = ind.astype(jnp.int32)

    e16 = jnp.arange(2 ** G)
    gtab = jnp.concatenate(
        [w[(e16 >> (G - 1 - j)) & 1] for j in range(G)], axis=1)
    gtab = jnp.tile(gtab, (NW, 1))
    idx = ind32[:SC_BATCH].reshape(B_G, G)
    gidx = jnp.zeros((B_G,), jnp.int32)
    for j in range(G):
        gidx = gidx * 2 + idx[:, j]
    gidx = gidx + (2 ** G) * (jnp.arange(B_G, dtype=jnp.int32) // BPW)
    big = _sc_embed(gtab, gidx).reshape(BATCH, D2)

    expand = jnp.einsum(
        "fg,d->fgd", jnp.eye(N_FIELDS, dtype=jnp.float32),
        jnp.ones((EMB,), jnp.float32)).reshape(N_FIELDS, D2)
    difft = jnp.tile(w[1] - w[0], (1, N_FIELDS)).reshape(1, D2)
    w0t = jnp.tile(w[0], (1, N_FIELDS)).reshape(1, D2)
    out = pl.pallas_call(
        _tc_body,
        grid=((BATCH - SC_BATCH) // TC_BB,),
        in_specs=[
            pl.BlockSpec((TC_BB, N_FIELDS), lambda i: (i + TC_OFF, 0)),
            pl.BlockSpec((N_FIELDS, D2), lambda i: (0, 0)),
            pl.BlockSpec((1, D2), lambda i: (0, 0)),
            pl.BlockSpec((1, D2), lambda i: (0, 0)),
            pl.BlockSpec(memory_space=pl.ANY),
        ],
        out_specs=pl.BlockSpec((TC_BB, D2), lambda i: (i + TC_OFF, 0)),
        out_shape=jax.ShapeDtypeStruct((BATCH, D2), jnp.float32),
        input_output_aliases={4: 0},
    )(ind32, expand, difft, w0t, big)

    return out.reshape(BATCH, N_FIELDS, EMB)


# speedup vs baseline: 1.0088x; 1.0088x over previous
import functools

import jax
import jax.numpy as jnp
from jax import lax
from jax.experimental import pallas as pl
from jax.experimental.pallas import tpu as pltpu
from jax.experimental.pallas import tpu_sc as plsc

BATCH = 16384
N_FIELDS = 26
EMB = 64
D2 = N_FIELDS * EMB
G = 4
GD = G * EMB
NC, NS = 2, 16
NW = NC * NS
CHUNK = 104
NBUF = 4
B_G_TOT = BATCH * N_FIELDS // G

SC_BATCH = 8192
B_G = SC_BATCH * N_FIELDS // G
BPW = B_G // NW
NCHUNK = BPW // CHUNK
TC_BB = 1024
TC_OFF = SC_BATCH // TC_BB

_mesh = plsc.VectorSubcoreMesh(core_axis_name="c", subcore_axis_name="s")


@functools.partial(
    pl.kernel,
    mesh=_mesh,
    out_type=jax.ShapeDtypeStruct((B_G_TOT, GD), jnp.float32),
    scratch_types=(
        [pltpu.VMEM((BPW,), jnp.int32)]
        + [pltpu.VMEM((CHUNK, GD), jnp.float32) for _ in range(NBUF)]
        + [pltpu.SemaphoreType.DMA for _ in range(2 * NBUF)]
    ),
)
def _sc_embed(table_hbm, idx_hbm, out_hbm, idx_v, *bufs):
    rows = bufs[:NBUF]
    sg = bufs[NBUF:2 * NBUF]
    sw = bufs[2 * NBUF:]
    wid = lax.axis_index("s") * NC + lax.axis_index("c")
    base0 = wid * BPW

    pltpu.sync_copy(idx_hbm.at[pl.ds(base0, BPW)], idx_v)

    def start_gather(k):
        b = k % NBUF
        return pltpu.async_copy(
            table_hbm.at[idx_v.at[pl.ds(k * CHUNK, CHUNK)]],
            rows[b], sg[b])

    def start_write(k):
        b = k % NBUF
        return pltpu.async_copy(
            rows[b], out_hbm.at[pl.ds(base0 + k * CHUNK, CHUNK)],
            sw[b])

    g = {k: start_gather(k) for k in range(NBUF - 1)}
    w = {}
    for k in range(NCHUNK):
        if k + NBUF - 1 < NCHUNK:
            if k >= 1:
                w[k - 1].wait()
            g[k + NBUF - 1] = start_gather(k + NBUF - 1)
        g[k].wait()
        w[k] = start_write(k)
    for k in range(max(0, NCHUNK - NBUF), NCHUNK):
        w[k].wait()


def _tc_body(ind_ref, e_ref, difft_ref, w0t_ref, buf_ref, out_ref):
    del buf_ref
    indf = ind_ref[...].astype(jnp.float32)
    rep = jnp.dot(indf, e_ref[...], preferred_element_type=jnp.float32)
    out_ref[...] = rep * difft_ref[...] + w0t_ref[...]


def kernel(ind, ind_emb_weight):
    w = ind_emb_weight
    ind32 = ind.astype(jnp.int32)

    e16 = jnp.arange(2 ** G)
    gtab = jnp.concatenate(
        [w[(e16 >> (G - 1 - j)) & 1] for j in range(G)], axis=1)
    gtab = jnp.tile(gtab, (NW, 1))
    idx = ind32[:SC_BATCH].reshape(B_G, G)
    gidx = jnp.zeros((B_G,), jnp.int32)
    for j in range(G):
        gidx = gidx * 2 + idx[:, j]
    gidx = gidx + (2 ** G) * (jnp.arange(B_G, dtype=jnp.int32) // BPW)
    big = _sc_embed(gtab, gidx).reshape(BATCH, D2)

    expand = jnp.einsum(
        "fg,d->fgd", jnp.eye(N_FIELDS, dtype=jnp.float32),
        jnp.ones((EMB,), jnp.float32)).reshape(N_FIELDS, D2)
    difft = jnp.tile(w[1] - w[0], (1, N_FIELDS)).reshape(1, D2)
    w0t = jnp.tile(w[0], (1, N_FIELDS)).reshape(1, D2)
    out = pl.pallas_call(
        _tc_body,
        grid=((BATCH - SC_BATCH) // TC_BB,),
        in_specs=[
            pl.BlockSpec((TC_BB, N_FIELDS), lambda i: (i + TC_OFF, 0)),
            pl.BlockSpec((N_FIELDS, D2), lambda i: (0, 0)),
            pl.BlockSpec((1, D2), lambda i: (0, 0)),
            pl.BlockSpec((1, D2), lambda i: (0, 0)),
            pl.BlockSpec(memory_space=pl.ANY),
        ],
        out_specs=pl.BlockSpec((TC_BB, D2), lambda i: (i + TC_OFF, 0)),
        out_shape=jax.ShapeDtypeStruct((BATCH, D2), jnp.float32),
        input_output_aliases={4: 0},
    )(ind32, expand, difft, w0t, big)

    return out.reshape(BATCH, N_FIELDS, EMB)
